# Initial kernel scaffold; baseline (speedup 1.0000x reference)
import jax
import jax.numpy as jnp
from jax.experimental import pallas as pl


def kernel(H_src, H_dst, HBar, edge_index, W, b):
    raise NotImplementedError("write your pallas kernel here")


# speedup vs baseline: 6.1897x; 6.1897x over previous
import functools

import jax
import jax.numpy as jnp
from jax import lax
from jax.experimental import pallas as pl
from jax.experimental.pallas import tpu as pltpu
from jax.experimental.pallas import tpu_sc as plsc

N = 10000
NP = 10240
E = 320000
D = 128
OUTF = 128
ALPHA = 0.1

NC = 2
NS = 16
NW = NC * NS
EPT = E // NW
K = 80
NCHUNK = EPT // K
RPT = NP // NS
RZ = 128


def _sc_body(hsrc, src, dst, p_out, c_out,
             src_v, dst_v, rows_v, ones_v, za_v, zc_v, accum_sh, cnt_sh, sem):
    cid = lax.axis_index("c")
    sid = lax.axis_index("s")

    def init_ones(i, carry):
        ones_v[pl.ds(i * 16, 16)] = jnp.ones((16,), jnp.float32)
        return carry

    lax.fori_loop(0, K // 16, init_ones, 0)

    def init_zeros(i, carry):
        def inner(j, c2):
            za_v[i, pl.ds(j * 16, 16)] = jnp.zeros((16,), jnp.float32)
            return c2
        lax.fori_loop(0, D // 16, inner, 0)
        return carry

    lax.fori_loop(0, RZ, init_zeros, 0)

    def init_zc(i, carry):
        zc_v[pl.ds(i * 16, 16)] = jnp.zeros((16,), jnp.float32)
        return carry

    lax.fori_loop(0, RPT // 16, init_zc, 0)

    r0 = sid * RPT
    for kk in range(RPT // RZ):
        pltpu.sync_copy(za_v, accum_sh.at[pl.ds(r0 + kk * RZ, RZ)])
    pltpu.sync_copy(zc_v, cnt_sh.at[pl.ds(r0, RPT)])
    plsc.subcore_barrier()

    ebase = (cid * NS + sid) * EPT

    def chunk(i, carry):
        base = ebase + i * K
        pltpu.sync_copy(src.at[pl.ds(base, K)], src_v)
        pltpu.sync_copy(dst.at[pl.ds(base, K)], dst_v)
        pltpu.async_copy(hsrc.at[src_v], rows_v, sem).wait()
        pltpu.sync_copy(rows_v, accum_sh.at[dst_v], add=True)
        pltpu.sync_copy(ones_v, cnt_sh.at[dst_v], add=True)
        return carry

    lax.fori_loop(0, NCHUNK, chunk, 0)
    plsc.subcore_barrier()

    pltpu.sync_copy(accum_sh.at[pl.ds(r0, RPT)], p_out.at[cid, pl.ds(r0, RPT)])
    pltpu.sync_copy(cnt_sh.at[pl.ds(r0, RPT)], c_out.at[cid, pl.ds(r0, RPT)])


@jax.jit
def _sc_segment_sum(hsrc, src, dst):
    mesh = plsc.VectorSubcoreMesh(core_axis_name="c", subcore_axis_name="s")
    return pl.kernel(
        _sc_body,
        out_type=(
            jax.ShapeDtypeStruct((NC, NP, D), jnp.float32),
            jax.ShapeDtypeStruct((NC, NP), jnp.float32),
        ),
        mesh=mesh,
        scratch_types=[
            pltpu.VMEM((K,), jnp.int32),
            pltpu.VMEM((K,), jnp.int32),
            pltpu.VMEM((K, D), jnp.float32),
            pltpu.VMEM((K,), jnp.float32),
            pltpu.VMEM((RZ, D), jnp.float32),
            pltpu.VMEM((RPT,), jnp.float32),
            pltpu.VMEM_SHARED((NP, D), jnp.float32),
            pltpu.VMEM_SHARED((NP,), jnp.float32),
            pltpu.SemaphoreType.DMA,
        ],
    )(hsrc, src, dst)


def _tc_body(hdst_ref, hbar_ref, p_ref, c_ref, w1_ref, w2_ref, b_ref,
             h_ref, hn_ref):
    s = p_ref[0] + p_ref[1]
    cnt = c_ref[0] + c_ref[1]
    deg = jnp.maximum(cnt, 1.0)
    hd = s / deg[:, None]
    hn = (1.0 - ALPHA) * (hbar_ref[...] - hd) + hd
    hn_ref[...] = hn
    acc = jnp.dot(hdst_ref[...], w1_ref[...], preferred_element_type=jnp.float32)
    acc = acc + jnp.dot(hn, w2_ref[...], preferred_element_type=jnp.float32)
    h_ref[...] = jnp.maximum(acc + b_ref[...], 0.0)


@jax.jit
def _tc_combine(hdst, hbar, p, c, w1, w2, b2d):
    R = 1024
    grid = (NP // R,)
    return pl.pallas_call(
        _tc_body,
        grid=grid,
        in_specs=[
            pl.BlockSpec((R, D), lambda i: (i, 0)),
            pl.BlockSpec((R, D), lambda i: (i, 0)),
            pl.BlockSpec((NC, R, D), lambda i: (0, i, 0)),
            pl.BlockSpec((NC, R), lambda i: (0, i)),
            pl.BlockSpec((D, OUTF), lambda i: (0, 0)),
            pl.BlockSpec((D, OUTF), lambda i: (0, 0)),
            pl.BlockSpec((1, OUTF), lambda i: (0, 0)),
        ],
        out_specs=[
            pl.BlockSpec((R, OUTF), lambda i: (i, 0)),
            pl.BlockSpec((R, D), lambda i: (i, 0)),
        ],
        out_shape=[
            jax.ShapeDtypeStruct((NP, OUTF), jnp.float32),
            jax.ShapeDtypeStruct((NP, D), jnp.float32),
        ],
    )(hdst, hbar, p, c, w1, w2, b2d)


def kernel(H_src, H_dst, HBar, edge_index, W, b):
    src = edge_index[0].astype(jnp.int32)
    dst = edge_index[1].astype(jnp.int32)
    p, c = _sc_segment_sum(H_src, src, dst)
    w1 = W[:, :D].T
    w2 = W[:, D:].T
    hdst_p = jnp.zeros((NP, D), jnp.float32).at[:N].set(H_dst)
    hbar_p = jnp.zeros((NP, D), jnp.float32).at[:N].set(HBar)
    h, hn = _tc_combine(hdst_p, hbar_p, p, c, w1, w2, b.reshape(1, OUTF))
    return (h[:N], hn[:N])
